# baseline (speedup 1.0000x reference)
import jax
import jax.numpy as jnp
from jax.experimental import pallas as pl
from jax.experimental.pallas import tpu as pltpu
from functools import partial

_B, _H, _W = 4, 224, 224
_NV = 1024
_P = 16
_SIGMA = 0.05
_N = _B * _H * _W
_C = 28672
_NH = 4
_HC = _C // _NH
_NBLK = _N // _C


def _soft_w_t(v_row, grid_col, inv2s2):
    d = v_row - grid_col
    w = jnp.exp(-(d * d) * inv2s2)
    s = jnp.sum(w, axis=0, keepdims=True)
    return w / (s + 1e-12)


def _hist_kernel(seg_ref, y_ref, x_ref, g1_ref, g2_ref, out_ref, acc_ref):
    i = pl.program_id(0)
    gi = jax.lax.broadcasted_iota(jnp.int32, (_P, 1), 0)
    grid_col = gi.astype(jnp.float32) * (2.0 / (_P - 1)) - 1.0
    inv2s2 = 1.0 / (2.0 * _SIGMA * _SIGMA)

    @pl.when(i == 0)
    def _():
        acc_ref[...] = jnp.zeros_like(acc_ref)

    iota_col = jax.lax.broadcasted_iota(jnp.int32, (_NV // 2, 1), 0)
    step_s = None
    step_d = None
    for h in range(_NH):
        sl = slice(h * _HC, (h + 1) * _HC)
        v0 = y_ref[0][:, sl] * (2.0 / _H) - 1.0
        v1 = x_ref[0][:, sl] * (2.0 / _W) - 1.0
        v2 = g1_ref[0][:, sl]
        v3 = g2_ref[0][:, sl]

        wa0 = _soft_w_t(v0, grid_col, inv2s2).astype(jnp.bfloat16)
        wb0 = _soft_w_t(v1, grid_col, inv2s2).astype(jnp.bfloat16)
        wa1 = _soft_w_t(v2, grid_col, inv2s2).astype(jnp.bfloat16)
        wb1 = _soft_w_t(v3, grid_col, inv2s2).astype(jnp.bfloat16)

        j0 = (wa0[:, None, :] * wb0[None, :, :]).reshape(_P * _P, _HC)
        j1 = (wa1[:, None, :] * wb1[None, :, :]).reshape(_P * _P, _HC)
        j = jnp.concatenate([j0, j1], axis=0)

        seg_row = seg_ref[0][:, sl]
        lo9 = seg_row & (_NV // 2 - 1)
        oh_s = (lo9 == iota_col).astype(jnp.bfloat16)
        sgn = (1 - 2 * (seg_row >> 9)).astype(jnp.bfloat16)
        oh_d = oh_s * sgn

        ds_ = jax.lax.dot_general(j, oh_s, (((1,), (1,)), ((), ())),
                                  preferred_element_type=jnp.float32)
        dd_ = jax.lax.dot_general(j, oh_d, (((1,), (1,)), ((), ())),
                                  preferred_element_type=jnp.float32)
        step_s = ds_ if step_s is None else step_s + ds_
        step_d = dd_ if step_d is None else step_d + dd_

    acc_ref[:, : _NV // 2] += step_s
    acc_ref[:, _NV // 2 :] += step_d

    @pl.when(i == _NBLK - 1)
    def _():
        s_acc = acc_ref[:, : _NV // 2]
        d_acc = acc_ref[:, _NV // 2 :]
        acc = jnp.concatenate([(s_acc + d_acc) * 0.5,
                               (s_acc - d_acc) * 0.5], axis=1)
        sizes = jnp.sum(acc[: _P * _P, :], axis=0, keepdims=True)
        out_ref[...] = acc * (4.0 / sizes)


def kernel(seg, byx, gfeat):
    seg_b = seg.reshape(-1).reshape(_NBLK, 1, _C)
    yf = byx[1].astype(jnp.float32).reshape(_NBLK, 1, _C)
    xf = byx[2].astype(jnp.float32).reshape(_NBLK, 1, _C)
    g1 = gfeat[:, 0, :, :].reshape(-1).reshape(_NBLK, 1, _C)
    g2 = gfeat[:, 1, :, :].reshape(-1).reshape(_NBLK, 1, _C)

    row_spec = pl.BlockSpec((1, 1, _C), lambda i: (i, 0, 0))
    out = pl.pallas_call(
        _hist_kernel,
        grid=(_NBLK,),
        in_specs=[
            pl.BlockSpec((1, 1, _C), lambda i: (i, 0, 0)),
            row_spec, row_spec, row_spec, row_spec,
        ],
        out_specs=pl.BlockSpec((2 * _P * _P, _NV), lambda i: (0, 0)),
        out_shape=jax.ShapeDtypeStruct((2 * _P * _P, _NV), jnp.float32),
        scratch_shapes=[pltpu.VMEM((2 * _P * _P, _NV), jnp.float32)],
        compiler_params=pltpu.CompilerParams(
            dimension_semantics=("arbitrary",),
        ),
    )(seg_b, yf, xf, g1, g2)

    return out.reshape(2, _P, _P, _NV).transpose(3, 0, 1, 2)


# speedup vs baseline: 1.0022x; 1.0022x over previous
import jax
import jax.numpy as jnp
from jax.experimental import pallas as pl
from jax.experimental.pallas import tpu as pltpu

_B, _H, _W = 4, 224, 224
_NV = 1024
_P = 16
_SIGMA = 0.05
_N = _B * _H * _W
_C = 28672
_NH = 4
_HC = _C // _NH
_NBLK = _N // _C


def _soft_w_t(v_row, grid_col, inv2s2):
    d = v_row - grid_col
    w = jnp.exp(-(d * d) * inv2s2)
    s = jnp.sum(w, axis=0, keepdims=True)
    return w / (s + 1e-12)


def _hist_kernel(seg_ref, y_ref, x_ref, g1_ref, g2_ref, out_ref, acc_ref):
    i = pl.program_id(0)
    gi = jax.lax.broadcasted_iota(jnp.int32, (_P, 1), 0)
    grid_col = gi.astype(jnp.float32) * (2.0 / (_P - 1)) - 1.0
    inv2s2 = 1.0 / (2.0 * _SIGMA * _SIGMA)

    @pl.when(i == 0)
    def _():
        acc_ref[...] = jnp.zeros_like(acc_ref)

    iota_col = jax.lax.broadcasted_iota(jnp.int32, (_NV, 1), 0)
    step = None
    for h in range(_NH):
        sl = slice(h * _HC, (h + 1) * _HC)
        v0 = y_ref[0][:, sl] * (2.0 / _H) - 1.0
        v1 = x_ref[0][:, sl] * (2.0 / _W) - 1.0
        v2 = g1_ref[0][:, sl]
        v3 = g2_ref[0][:, sl]

        wa0 = _soft_w_t(v0, grid_col, inv2s2).astype(jnp.bfloat16)
        wb0 = _soft_w_t(v1, grid_col, inv2s2).astype(jnp.bfloat16)
        wa1 = _soft_w_t(v2, grid_col, inv2s2).astype(jnp.bfloat16)
        wb1 = _soft_w_t(v3, grid_col, inv2s2).astype(jnp.bfloat16)

        j0 = (wa0[:, None, :] * wb0[None, :, :]).reshape(_P * _P, _HC)
        j1 = (wa1[:, None, :] * wb1[None, :, :]).reshape(_P * _P, _HC)
        j = jnp.concatenate([j0, j1], axis=0)

        seg_row = seg_ref[0][:, sl]
        onehot_t = (seg_row == iota_col).astype(jnp.bfloat16)

        d = jax.lax.dot_general(j, onehot_t, (((1,), (1,)), ((), ())),
                                preferred_element_type=jnp.float32)
        step = d if step is None else step + d

    acc_ref[...] += step

    @pl.when(i == _NBLK - 1)
    def _():
        acc = acc_ref[...]
        sizes = jnp.sum(acc[: _P * _P, :], axis=0, keepdims=True)
        out_ref[...] = acc * (4.0 / sizes)


def kernel(seg, byx, gfeat):
    seg_b = seg.reshape(-1).reshape(_NBLK, 1, _C)
    yf = byx[1].astype(jnp.float32).reshape(_NBLK, 1, _C)
    xf = byx[2].astype(jnp.float32).reshape(_NBLK, 1, _C)
    g1 = gfeat[:, 0, :, :].reshape(-1).reshape(_NBLK, 1, _C)
    g2 = gfeat[:, 1, :, :].reshape(-1).reshape(_NBLK, 1, _C)

    row_spec = pl.BlockSpec((1, 1, _C), lambda i: (i, 0, 0))
    out = pl.pallas_call(
        _hist_kernel,
        grid=(_NBLK,),
        in_specs=[
            pl.BlockSpec((1, 1, _C), lambda i: (i, 0, 0)),
            row_spec, row_spec, row_spec, row_spec,
        ],
        out_specs=pl.BlockSpec((2 * _P * _P, _NV), lambda i: (0, 0)),
        out_shape=jax.ShapeDtypeStruct((2 * _P * _P, _NV), jnp.float32),
        scratch_shapes=[pltpu.VMEM((2 * _P * _P, _NV), jnp.float32)],
        compiler_params=pltpu.CompilerParams(
            dimension_semantics=("arbitrary",),
        ),
    )(seg_b, yf, xf, g1, g2)

    return out.reshape(2, _P, _P, _NV).transpose(3, 0, 1, 2)
